# baseline (speedup 1.0000x reference)
import functools

import jax
import jax.numpy as jnp
from jax import lax
from jax.experimental import pallas as pl
from jax.experimental.pallas import tpu as pltpu
from jax.experimental.pallas import tpu_sc as plsc

N = 10000
E = 320000
F_IN = 128
H = 64
C = 40
CP = 48

NC = 2
NS = 16
NW = NC * NS
B = 128
BLK = E // B
BPW = BLK // NW
NTAIL = BLK - BPW * NW
NBUF = 4
RB = 128
NRC = N // RB
NREM = N - NRC * RB


def _mm1_body(x_ref, w_ref, b_ref, o_ref):
    o_ref[...] = (
        jnp.dot(x_ref[...], w_ref[...], preferred_element_type=jnp.float32)
        + b_ref[...]
    )


def _mm2_body(p0_ref, p1_ref, w_ref, b_ref, o_ref):
    h = jnp.maximum(p0_ref[...] + p1_ref[...], 0.0)
    o_ref[...] = (
        jnp.dot(h, w_ref[...], preferred_element_type=jnp.float32) + b_ref[...]
    )


def _lsm_body(q0_ref, q1_ref, o_ref):
    logits = (q0_ref[...] + q1_ref[...])[:, :C]
    m = jnp.max(logits, axis=1, keepdims=True)
    ex = jnp.exp(logits - m)
    lse = jnp.log(jnp.sum(ex, axis=1, keepdims=True)) + m
    o_ref[...] = logits - lse


def _make_spmm(D):
    mesh = plsc.VectorSubcoreMesh(core_axis_name="c", subcore_axis_name="s")

    @functools.partial(
        pl.kernel,
        out_type=[
            jax.ShapeDtypeStruct((N, D), jnp.float32),
            jax.ShapeDtypeStruct((N, D), jnp.float32),
        ],
        mesh=mesh,
        scratch_types=[
            pltpu.VMEM((BPW, 2, B), jnp.int32),
            pltpu.VMEM((2, B), jnp.int32),
            pltpu.VMEM((BPW, B), jnp.float32),
            pltpu.VMEM((B,), jnp.float32),
            [pltpu.VMEM((B, D), jnp.float32)] * NBUF,
            pltpu.VMEM_SHARED((N, D), jnp.float32),
            [pltpu.SemaphoreType.DMA] * NBUF,
            [pltpu.SemaphoreType.DMA] * NBUF,
        ],
        compiler_params=pltpu.CompilerParams(use_tc_tiling_on_sc=False),
    )
    def spmm(h_hbm, ei_hbm, w_hbm, p0_hbm, p1_hbm,
             eiv, eit, wv, wvt, rows, acc, gsem, ssem):
        cid = lax.axis_index("c")
        sid = lax.axis_index("s")
        wid = sid * NC + cid

        zero16 = jnp.zeros((16,), jnp.float32)
        for e in range(B):
            for j in range(D // 16):
                rows[0][e, pl.ds(j * 16, 16)] = zero16
        for i in range(5):
            cz = sid * 5 + i

            @pl.when(cz < NRC)
            def _():
                pltpu.sync_copy(rows[0], acc.at[pl.ds(cz * RB, RB)])

        @pl.when(sid == 0)
        def _():
            pltpu.sync_copy(rows[0].at[pl.ds(0, NREM)],
                            acc.at[pl.ds(NRC * RB, NREM)])

        pltpu.sync_copy(ei_hbm.at[pl.ds(wid * BPW, BPW)], eiv)
        pltpu.sync_copy(w_hbm.at[pl.ds(wid * BPW, BPW)], wv)

        @pl.when(wid < NTAIL)
        def _():
            pltpu.sync_copy(ei_hbm.at[NW * BPW + wid], eit)
            pltpu.sync_copy(w_hbm.at[NW * BPW + wid], wvt)

        plsc.subcore_barrier()

        def gather(k, i):
            return pltpu.make_async_copy(
                h_hbm.at[eiv.at[k, 1]], rows[i], gsem[i])

        def scatter(k, i):
            return pltpu.make_async_copy(rows[i], acc.at[eiv.at[k, 0]],
                                         ssem[i])

        def scale(k, i):
            @pl.loop(0, B // 16)
            def _(g):
                wvec = wv[k, pl.ds(g * 16, 16)]
                for el in range(16):
                    w = wvec[el]
                    for j in range(D // 16):
                        sl = pl.ds(j * 16, 16)
                        rows[i][g * 16 + el, sl] = rows[i][g * 16 + el, sl] * w

        for i in range(NBUF):
            gather(i, i).start()

        def block_body(t, carry):
            for i in range(NBUF):
                k = t * NBUF + i
                gather(k, i).wait()
                scale(k, i)
                scatter(k, i).start(add=True)
            for i in range(NBUF):
                k2 = t * NBUF + i + NBUF
                scatter(k2, i).wait()

                @pl.when(k2 < BPW)
                def _():
                    gather(k2, i).start()
            return carry

        lax.fori_loop(0, BPW // NBUF, block_body, 0)

        for i in range(BPW % NBUF):
            ke = (BPW // NBUF) * NBUF + i
            gather(ke, i).wait()
            scale(ke, i)
            scatter(ke, i).start(add=True)
            scatter(ke, i).wait()

        @pl.when(wid < NTAIL)
        def _():
            tb = NBUF - 1
            pltpu.make_async_copy(
                h_hbm.at[eit.at[1]], rows[tb], gsem[tb]).start()
            pltpu.make_async_copy(
                h_hbm.at[eit.at[1]], rows[tb], gsem[tb]).wait()

            @pl.loop(0, B // 16)
            def _(g):
                wvec = wvt[pl.ds(g * 16, 16)]
                for el in range(16):
                    w = wvec[el]
                    for j in range(D // 16):
                        sl = pl.ds(j * 16, 16)
                        rows[tb][g * 16 + el, sl] = (
                            rows[tb][g * 16 + el, sl] * w)

            pltpu.make_async_copy(
                rows[tb], acc.at[eit.at[0]], ssem[tb]).start(add=True)
            pltpu.make_async_copy(
                rows[tb], acc.at[eit.at[0]], ssem[tb]).wait()

        plsc.subcore_barrier()

        def writeout(dst_hbm):
            for i in range(5):
                cz = sid * 5 + i

                @pl.when(cz < NRC)
                def _():
                    sl = pl.ds(cz * RB, RB)
                    pltpu.sync_copy(acc.at[sl], dst_hbm.at[sl])

            @pl.when(sid == 0)
            def _():
                sl = pl.ds(NRC * RB, NREM)
                pltpu.sync_copy(acc.at[sl], dst_hbm.at[sl])

        @pl.when(cid == 0)
        def _():
            writeout(p0_hbm)

        @pl.when(cid == 1)
        def _():
            writeout(p1_hbm)

    return spmm


_spmm_h = _make_spmm(H)
_spmm_c = _make_spmm(CP)


def kernel(x, edge_index, edge_weight, W1, b1, W2, b2):
    ei = edge_index.T.reshape(BLK, B, 2).swapaxes(1, 2)
    w = edge_weight.reshape(BLK, B)

    h = pl.pallas_call(
        _mm1_body,
        out_shape=jax.ShapeDtypeStruct((N, H), jnp.float32),
    )(x, W1.T, b1.reshape(1, H))

    p0, p1 = _spmm_h(h, ei, w)

    w2p = jnp.pad(W2.T, ((0, 0), (0, CP - C)))
    b2p = jnp.pad(b2, (0, CP - C)).reshape(1, CP)
    h2 = pl.pallas_call(
        _mm2_body,
        out_shape=jax.ShapeDtypeStruct((N, CP), jnp.float32),
    )(p0, p1, w2p, b2p)

    q0, q1 = _spmm_c(h2, ei, w)

    out = pl.pallas_call(
        _lsm_body,
        out_shape=jax.ShapeDtypeStruct((N, C), jnp.float32),
    )(q0, q1)
    return out


# speedup vs baseline: 1.3244x; 1.3244x over previous
import functools

import jax
import jax.numpy as jnp
from jax import lax
from jax.experimental import pallas as pl
from jax.experimental.pallas import tpu as pltpu
from jax.experimental.pallas import tpu_sc as plsc

N = 10000
E = 320000
F_IN = 128
H = 64
C = 40
CP = 48

NC = 2
NS = 16
NW = NC * NS
B = 128
BLK = E // B
BPW = BLK // NW
NTAIL = BLK - BPW * NW
NBUF = 3
RB = 128
NRC = N // RB
NREM = N - NRC * RB


def _mm1_body(x_ref, w_ref, b_ref, o_ref):
    o_ref[...] = (
        jnp.dot(x_ref[...], w_ref[...], preferred_element_type=jnp.float32)
        + b_ref[...]
    )


def _mm2_body(p0_ref, p1_ref, w_ref, b_ref, o_ref):
    h = jnp.maximum(p0_ref[...] + p1_ref[...], 0.0)
    o_ref[...] = (
        jnp.dot(h, w_ref[...], preferred_element_type=jnp.float32) + b_ref[...]
    )


def _lsm_body(q0_ref, q1_ref, o_ref):
    logits = (q0_ref[...] + q1_ref[...])[:, :C]
    m = jnp.max(logits, axis=1, keepdims=True)
    ex = jnp.exp(logits - m)
    lse = jnp.log(jnp.sum(ex, axis=1, keepdims=True)) + m
    o_ref[...] = logits - lse


def _make_spmm(D):
    mesh = plsc.VectorSubcoreMesh(core_axis_name="c", subcore_axis_name="s")

    @functools.partial(
        pl.kernel,
        out_type=[
            jax.ShapeDtypeStruct((N, D), jnp.float32),
            jax.ShapeDtypeStruct((N, D), jnp.float32),
        ],
        mesh=mesh,
        scratch_types=[
            pltpu.VMEM((BPW, 2, B), jnp.int32),
            pltpu.VMEM((2, B), jnp.int32),
            pltpu.VMEM((BPW, B), jnp.float32),
            pltpu.VMEM((B,), jnp.float32),
            [pltpu.VMEM((B, D), jnp.float32)] * NBUF,
            pltpu.VMEM_SHARED((N, D), jnp.float32),
            [pltpu.SemaphoreType.DMA] * NBUF,
            [pltpu.SemaphoreType.DMA] * NBUF,
        ],
        compiler_params=pltpu.CompilerParams(use_tc_tiling_on_sc=False),
    )
    def spmm(h_hbm, ei_hbm, w_hbm, p0_hbm, p1_hbm,
             eiv, eit, wv, wvt, rows, acc, gsem, ssem):
        cid = lax.axis_index("c")
        sid = lax.axis_index("s")
        wid = sid * NC + cid

        zero16 = jnp.zeros((16,), jnp.float32)
        for e in range(B):
            for j in range(D // 16):
                rows[0][e, pl.ds(j * 16, 16)] = zero16
        for i in range(5):
            cz = sid * 5 + i

            @pl.when(cz < NRC)
            def _():
                pltpu.sync_copy(rows[0], acc.at[pl.ds(cz * RB, RB)])

        @pl.when(sid == 0)
        def _():
            pltpu.sync_copy(rows[0].at[pl.ds(0, NREM)],
                            acc.at[pl.ds(NRC * RB, NREM)])

        pltpu.sync_copy(ei_hbm.at[pl.ds(wid * BPW, BPW)], eiv)
        pltpu.sync_copy(w_hbm.at[pl.ds(wid * BPW, BPW)], wv)

        @pl.when(wid < NTAIL)
        def _():
            pltpu.sync_copy(ei_hbm.at[NW * BPW + wid], eit)
            pltpu.sync_copy(w_hbm.at[NW * BPW + wid], wvt)

        plsc.subcore_barrier()

        def gather(k, i):
            return pltpu.make_async_copy(
                h_hbm.at[eiv.at[k, 1]], rows[i], gsem[i])

        def scatter(k, i):
            return pltpu.make_async_copy(rows[i], acc.at[eiv.at[k, 0]],
                                         ssem[i])

        def scale(k, i):
            for g in range(B // 16):
                wvec = wv[k, pl.ds(g * 16, 16)]
                for el in range(16):
                    e = g * 16 + el
                    w = wvec[el]
                    for j in range(D // 16):
                        sl = pl.ds(j * 16, 16)
                        rows[i][e, sl] = rows[i][e, sl] * w

        for i in range(NBUF):
            gather(i, i).start()

        def block_body(t, carry):
            for i in range(NBUF):
                k = t * NBUF + i
                gather(k, i).wait()
                scale(k, i)
                scatter(k, i).start(add=True)
            for i in range(NBUF):
                k2 = t * NBUF + i + NBUF
                scatter(k2, i).wait()

                @pl.when(k2 < BPW)
                def _():
                    gather(k2, i).start()
            return carry

        lax.fori_loop(0, BPW // NBUF, block_body, 0)

        @pl.when(wid < NTAIL)
        def _():
            tb = NBUF - 1
            pltpu.make_async_copy(
                h_hbm.at[eit.at[1]], rows[tb], gsem[tb]).start()
            pltpu.make_async_copy(
                h_hbm.at[eit.at[1]], rows[tb], gsem[tb]).wait()

            for g in range(B // 16):
                wvec = wvt[pl.ds(g * 16, 16)]
                for el in range(16):
                    e = g * 16 + el
                    w = wvec[el]
                    for j in range(D // 16):
                        sl = pl.ds(j * 16, 16)
                        rows[tb][e, sl] = rows[tb][e, sl] * w

            pltpu.make_async_copy(
                rows[tb], acc.at[eit.at[0]], ssem[tb]).start(add=True)
            pltpu.make_async_copy(
                rows[tb], acc.at[eit.at[0]], ssem[tb]).wait()

        plsc.subcore_barrier()

        def writeout(dst_hbm):
            for i in range(5):
                cz = sid * 5 + i

                @pl.when(cz < NRC)
                def _():
                    sl = pl.ds(cz * RB, RB)
                    pltpu.sync_copy(acc.at[sl], dst_hbm.at[sl])

            @pl.when(sid == 0)
            def _():
                sl = pl.ds(NRC * RB, NREM)
                pltpu.sync_copy(acc.at[sl], dst_hbm.at[sl])

        @pl.when(cid == 0)
        def _():
            writeout(p0_hbm)

        @pl.when(cid == 1)
        def _():
            writeout(p1_hbm)

    return spmm


_spmm_h = _make_spmm(H)
_spmm_c = _make_spmm(CP)


def kernel(x, edge_index, edge_weight, W1, b1, W2, b2):
    ei = edge_index.T.reshape(BLK, B, 2).swapaxes(1, 2)
    w = edge_weight.reshape(BLK, B)

    h = pl.pallas_call(
        _mm1_body,
        out_shape=jax.ShapeDtypeStruct((N, H), jnp.float32),
    )(x, W1.T, b1.reshape(1, H))

    p0, p1 = _spmm_h(h, ei, w)

    w2p = jnp.pad(W2.T, ((0, 0), (0, CP - C)))
    b2p = jnp.pad(b2, (0, CP - C)).reshape(1, CP)
    h2 = pl.pallas_call(
        _mm2_body,
        out_shape=jax.ShapeDtypeStruct((N, CP), jnp.float32),
    )(p0, p1, w2p, b2p)

    q0, q1 = _spmm_c(h2, ei, w)

    out = pl.pallas_call(
        _lsm_body,
        out_shape=jax.ShapeDtypeStruct((N, C), jnp.float32),
    )(q0, q1)
    return out
